# Initial kernel scaffold; baseline (speedup 1.0000x reference)
import jax
import jax.numpy as jnp
from jax.experimental import pallas as pl


def kernel(tokens, word_vectors):
    raise NotImplementedError("write your pallas kernel here")


# speedup vs baseline: 3.2640x; 3.2640x over previous
import functools

import jax
import jax.numpy as jnp
from jax import lax
from jax.experimental import pallas as pl
from jax.experimental.pallas import tpu as pltpu
from jax.experimental.pallas import tpu_sc as plsc

VOCAB = 100000
EMB = 64
SCALE = 8.0

B, L = 4096, 200
B_FLAT = B * L
NC, NS = 2, 16
NW = NC * NS
PER_W = B_FLAT // NW
CHUNK = 128
NCHUNK = PER_W // CHUNK


def _scale_body(w_ref, o_ref):
    o_ref[...] = w_ref[...] * SCALE


def _scale_table(word_vectors):
    return pl.pallas_call(
        _scale_body,
        out_shape=jax.ShapeDtypeStruct((VOCAB, EMB), jnp.float32),
        grid=(50,),
        in_specs=[pl.BlockSpec((VOCAB // 50, EMB), lambda i: (i, 0))],
        out_specs=pl.BlockSpec((VOCAB // 50, EMB), lambda i: (i, 0)),
    )(word_vectors)


_MESH = plsc.VectorSubcoreMesh(core_axis_name="c", subcore_axis_name="s")


@functools.partial(
    pl.kernel,
    mesh=_MESH,
    out_type=jax.ShapeDtypeStruct((B_FLAT, EMB), jnp.float32),
    scratch_types=[
        pltpu.VMEM((NCHUNK, CHUNK), jnp.int32),
        pltpu.VMEM((CHUNK, EMB), jnp.float32),
        pltpu.SemaphoreType.DMA,
    ],
    compiler_params=pltpu.CompilerParams(use_tc_tiling_on_sc=False),
)
def _gather(table_hbm, idx_hbm, out_hbm, idx_v, rows_v, sem):
    wid = lax.axis_index("s") * NC + lax.axis_index("c")
    base = wid * PER_W
    pltpu.sync_copy(idx_hbm.at[wid], idx_v)

    def body(j, carry):
        pltpu.async_copy(table_hbm.at[idx_v.at[j]], rows_v, sem).wait()
        pltpu.sync_copy(rows_v, out_hbm.at[pl.ds(base + j * CHUNK, CHUNK)])
        return carry

    lax.fori_loop(0, NCHUNK, body, 0)


def kernel(tokens, word_vectors):
    table = _scale_table(word_vectors)
    idx = tokens.reshape(NW, NCHUNK, CHUNK)
    out = _gather(table, idx)
    return out.reshape(B, L, EMB)


# speedup vs baseline: 3.8636x; 1.1837x over previous
import functools

import jax
import jax.numpy as jnp
from jax import lax
from jax.experimental import pallas as pl
from jax.experimental.pallas import tpu as pltpu
from jax.experimental.pallas import tpu_sc as plsc

VOCAB = 100000
EMB = 64
SCALE = 8.0

B, L = 4096, 200
B_FLAT = B * L
NC, NS = 2, 16
NW = NC * NS
PER_W = B_FLAT // NW
CHUNK = 128
NCHUNK = PER_W // CHUNK


def _scale_body(w_ref, o_ref):
    o_ref[...] = w_ref[...] * SCALE


def _scale_table(word_vectors):
    return pl.pallas_call(
        _scale_body,
        out_shape=jax.ShapeDtypeStruct((VOCAB, EMB), jnp.float32),
        grid=(50,),
        in_specs=[pl.BlockSpec((VOCAB // 50, EMB), lambda i: (i, 0))],
        out_specs=pl.BlockSpec((VOCAB // 50, EMB), lambda i: (i, 0)),
    )(word_vectors)


_MESH = plsc.VectorSubcoreMesh(core_axis_name="c", subcore_axis_name="s")

NBUF = 8
LAG = 4
NGROUP = NCHUNK // NBUF


@functools.partial(
    pl.kernel,
    mesh=_MESH,
    out_type=jax.ShapeDtypeStruct((B_FLAT, EMB), jnp.float32),
    scratch_types=[
        pltpu.VMEM((NCHUNK, CHUNK), jnp.int32),
        pltpu.VMEM((NBUF, CHUNK, EMB), jnp.float32),
        pltpu.SemaphoreType.DMA((NBUF,)),
        pltpu.SemaphoreType.DMA((NBUF,)),
    ],
    compiler_params=pltpu.CompilerParams(use_tc_tiling_on_sc=False),
)
def _gather(table_hbm, idx_hbm, out_hbm, idx_v, rows, gsem, wsem):
    wid = lax.axis_index("s") * NC + lax.axis_index("c")
    base = wid * PER_W
    pltpu.sync_copy(idx_hbm.at[wid], idx_v)

    def issue_g(c, b):
        pltpu.async_copy(table_hbm.at[idx_v.at[c]], rows.at[b], gsem.at[b])

    def wait_g(c, b):
        pltpu.make_async_copy(
            table_hbm.at[idx_v.at[c]], rows.at[b], gsem.at[b]).wait()

    def issue_w(c, b):
        pltpu.async_copy(
            rows.at[b], out_hbm.at[pl.ds(base + c * CHUNK, CHUNK)], wsem.at[b])

    def wait_w(c, b):
        pltpu.make_async_copy(
            rows.at[b], out_hbm.at[pl.ds(base + c * CHUNK, CHUNK)],
            wsem.at[b]).wait()

    for j in range(NBUF):
        issue_g(j, j)
        if j >= LAG:
            wait_g(j - LAG, j - LAG)
            issue_w(j - LAG, j - LAG)

    def outer(g, carry):
        for b in range(NBUF):
            j = g * NBUF + b
            wait_w(j - NBUF, b)
            issue_g(j, b)
            b2 = (b + NBUF - LAG) % NBUF
            wait_g(j - LAG, b2)
            issue_w(j - LAG, b2)
        return carry

    lax.fori_loop(1, NGROUP, outer, 0)

    for j in range(NCHUNK, NCHUNK + LAG):
        b2 = (j - LAG) % NBUF
        wait_g(j - LAG, b2)
        issue_w(j - LAG, b2)
    for c in range(NCHUNK - NBUF, NCHUNK):
        wait_w(c, c % NBUF)


def kernel(tokens, word_vectors):
    table = _scale_table(word_vectors)
    idx = tokens.reshape(NW, NCHUNK, CHUNK)
    out = _gather(table, idx)
    return out.reshape(B, L, EMB)


# speedup vs baseline: 13.2576x; 3.4314x over previous
import functools

import jax
import jax.numpy as jnp
from jax import lax
from jax.experimental import pallas as pl
from jax.experimental.pallas import tpu as pltpu
from jax.experimental.pallas import tpu_sc as plsc

VOCAB = 100000
EMB = 64
SCALE = 8.0

B, L = 4096, 200
NC, NS = 2, 16
NW = NC * NS
BB = B // NW

NBUF = 4
LAG = 2
NGROUP = L // NBUF


_MESH = plsc.VectorSubcoreMesh(core_axis_name="c", subcore_axis_name="s")


@functools.partial(
    pl.kernel,
    mesh=_MESH,
    out_type=jax.ShapeDtypeStruct((L, 8, NW, 8, BB), jnp.float32),
    scratch_types=[
        pltpu.VMEM((L, BB), jnp.int32),
        pltpu.VMEM((NBUF, BB, EMB), jnp.float32),
        pltpu.VMEM((NBUF, 8, 8, BB + 1), jnp.float32),
        pltpu.SemaphoreType.DMA((NBUF,)),
        pltpu.SemaphoreType.DMA((NBUF,)),
    ],
    compiler_params=pltpu.CompilerParams(
        use_tc_tiling_on_sc=False, needs_layout_passes=False,
        disable_bounds_checks=True),
)
def _gather_t(table_hbm, idx_hbm, out_hbm, idx_v, rows, trows, gsem, wsem):
    wid = lax.axis_index("s") * NC + lax.axis_index("c")
    pltpu.sync_copy(idx_hbm.at[wid], idx_v)

    iota = lax.iota(jnp.int32, 16)
    iota_div8 = lax.shift_right_logical(iota, 3)
    iota_mod8_x128 = lax.shift_left(
        lax.bitwise_and(iota, jnp.full((16,), 7, jnp.int32)), 7)

    def issue_g(l, b):
        pltpu.async_copy(table_hbm.at[idx_v.at[l]], rows.at[b], gsem.at[b])

    def wait_g(l, b):
        pltpu.make_async_copy(
            table_hbm.at[idx_v.at[l]], rows.at[b], gsem.at[b]).wait()

    row_sel = [iota_div8 + 2 * k for k in range(EMB // 16)]
    sub_sel = lax.bitwise_and(iota, jnp.full((16,), 7, jnp.int32))
    zeros16 = jnp.full((16,), 0, jnp.int32)

    def transpose_scale(b):
        def rbody(r):
            col = zeros16 + r
            for k in range(EMB // 16):
                v = rows[b, r, pl.ds(16 * k, 16)]
                plsc.store_scatter(trows.at[b], [row_sel[k], sub_sel, col],
                                   v * SCALE)
        plsc.parallel_loop(0, BB, 1, unroll=2)(rbody)

    def issue_w(l, b):
        pltpu.async_copy(trows.at[b, :, :, pl.ds(0, BB)],
                         out_hbm.at[l, :, wid], wsem.at[b])

    def wait_w(l, b):
        pltpu.make_async_copy(trows.at[b, :, :, pl.ds(0, BB)],
                              out_hbm.at[l, :, wid], wsem.at[b]).wait()

    for j in range(NBUF):
        issue_g(j, j)
        if j >= LAG:
            wait_g(j - LAG, j - LAG)
            transpose_scale(j - LAG)
            issue_w(j - LAG, j - LAG)

    def outer(g, carry):
        for b in range(NBUF):
            j = g * NBUF + b
            wait_w(j - NBUF, b)
            issue_g(j, b)
            b2 = (b + NBUF - LAG) % NBUF
            wait_g(j - LAG, b2)
            transpose_scale(b2)
            issue_w(j - LAG, b2)
        return carry

    lax.fori_loop(1, NGROUP, outer, 0)

    for j in range(L, L + LAG):
        b2 = (j - LAG) % NBUF
        wait_g(j - LAG, b2)
        transpose_scale(b2)
        issue_w(j - LAG, b2)
    for l in range(L - NBUF, L):
        wait_w(l, l % NBUF)


def kernel(tokens, word_vectors):
    idx = tokens.reshape(NW, BB, L).transpose(0, 2, 1)
    out_t = _gather_t(word_vectors, idx)
    return out_t.transpose(2, 4, 0, 1, 3).reshape(B, L, EMB)
